# Initial kernel scaffold; baseline (speedup 1.0000x reference)
import jax
import jax.numpy as jnp
from jax.experimental import pallas as pl


def kernel(positions, colors, markers, anchors, n_jumps, targets, pre_Win, pre_bin, pre_Wh, pre_bh, pre_Wout, pre_bout, msg_Win, msg_bin, msg_Wh, msg_bh, msg_Wout, msg_bout, post_Win, post_bin, post_Wh, post_bh, post_Wout, post_bout, out_Win, out_bin, out_Wh, out_bh, out_Wout, out_bout, bn_gamma, bn_beta, lstm_W, lstm_b):
    raise NotImplementedError("write your pallas kernel here")


# speedup vs baseline: 17.9089x; 17.9089x over previous
import jax
import jax.numpy as jnp
import numpy as np
from jax.experimental import pallas as pl

_BS = 512
_N = 8
_H = 128
_STEPS = 8
_NA = 16
_ROWS = _BS * _N
_DIN = 2 + 8 + 8 + _NA + _N


def _dot(a, b):
    return jax.lax.dot_general(a, b, (((1,), (0,)), ((), ())),
                               preferred_element_type=jnp.float32)


def _mlp4(x, Win, bi, W0, b0, W1, b1, Wout, bo):
    x = jnp.maximum(_dot(x, Win) + bi, 0.0)
    x = jnp.maximum(_dot(x, W0) + b0, 0.0)
    x = jnp.maximum(_dot(x, W1) + b1, 0.0)
    return _dot(x, Wout) + bo


def _fwd_kernel(xin_ref, tgt_ref,
                preWin, prebin, preW0, preb0, preW1, preb1, preWout, prebout,
                msgWin, msgbin, msgW0, msgb0, msgW1, msgb1, msgWout, msgbout,
                postWin, postbin, postW0, postb0, postW1, postb1, postWout,
                postbout,
                outWin, outbin, outW0, outb0, outW1, outb1, outWout, outbout,
                gamma_ref, beta_ref, lstmW, lstmb,
                loss_ref, idx_ref):
    x = _mlp4(xin_ref[...], preWin[...], prebin[...], preW0[...], preb0[...],
              preW1[...], preb1[...], preWout[...], prebout[...])
    x0 = x
    c = jnp.zeros((_ROWS, _H), jnp.float32)
    h = jnp.zeros((_ROWS, _H), jnp.float32)
    gamma = gamma_ref[...]
    beta = beta_ref[...]
    tgt = tgt_ref[...]
    losses = []
    idxs = []
    for _ in range(_STEPS):
        x3 = x.reshape(_BS, _N, _H)
        xm = jnp.zeros((_ROWS, _H), jnp.float32)
        for i in range(_N):
            xi = jnp.broadcast_to(x3[:, i, :][:, None, :],
                                  (_BS, _N, _H)).reshape(_ROWS, _H)
            e = jnp.concatenate([xi, x], axis=1)
            m = _mlp4(e, msgWin[...], msgbin[...], msgW0[...], msgb0[...],
                      msgW1[...], msgb1[...], msgWout[...], msgbout[...])
            xm = xm + m
        t = jnp.concatenate([xm, x0], axis=1)
        x = _mlp4(t, postWin[...], postbin[...], postW0[...], postb0[...],
                  postW1[...], postb1[...], postWout[...], postbout[...])
        mean = jnp.mean(x, axis=0)
        var = jnp.var(x, axis=0)
        x = (x - mean) / jnp.sqrt(var + 1e-3) * gamma + beta
        gates = _dot(jnp.concatenate([x, h], axis=1), lstmW[...]) + lstmb[...]
        gi = gates[:, 0 * _H:1 * _H]
        gj = gates[:, 1 * _H:2 * _H]
        gf = gates[:, 2 * _H:3 * _H]
        go = gates[:, 3 * _H:4 * _H]
        c = jax.nn.sigmoid(gf + 1.0) * c + jax.nn.sigmoid(gi) * jnp.tanh(gj)
        h = jnp.tanh(c) * jax.nn.sigmoid(go)
        x = h
        xs = jnp.sum(x.reshape(_BS, _N, _H), axis=1)
        logits = _mlp4(xs, outWin[...], outbin[...], outW0[...], outb0[...],
                       outW1[...], outb1[...], outWout[...], outbout[...])
        mx = jnp.max(logits, axis=1, keepdims=True)
        ii = jax.lax.broadcasted_iota(jnp.int32, (_BS, _NA), 1)
        idxs.append(jnp.min(jnp.where(logits == mx, ii, _NA), axis=1))
        z = logits - mx
        lse = jnp.log(jnp.sum(jnp.exp(z), axis=1, keepdims=True))
        ce = -jnp.sum((z - lse) * tgt, axis=1)
        losses.append(jnp.mean(ce) / np.log(2.0))
    idx_ref[...] = jnp.stack(idxs, axis=0)
    loss_ref[...] = jnp.stack(losses).reshape(_STEPS, 1)


@jax.jit
def kernel(positions, colors, markers, anchors, n_jumps, targets,
           pre_Win, pre_bin, pre_Wh, pre_bh, pre_Wout, pre_bout,
           msg_Win, msg_bin, msg_Wh, msg_bh, msg_Wout, msg_bout,
           post_Win, post_bin, post_Wh, post_bh, post_Wout, post_bout,
           out_Win, out_bin, out_Wh, out_bh, out_Wout, out_bout,
           bn_gamma, bn_beta, lstm_W, lstm_b):
    pos = positions.reshape(_ROWS, 2)
    col = jax.nn.one_hot(colors, 8).reshape(_ROWS, 8)
    mar = jax.nn.one_hot(markers, 8).reshape(_ROWS, 8)
    q = jnp.concatenate([jax.nn.one_hot(anchors, _NA),
                         jax.nn.one_hot(n_jumps, _N)], axis=1)
    q = jnp.repeat(q, _N, axis=0, total_repeat_length=_ROWS)
    xin = jnp.concatenate([pos, col, mar, q], axis=1)
    xin = jnp.pad(xin, ((0, 0), (0, _H - _DIN)))
    preWin = jnp.pad(pre_Win, ((0, _H - _DIN), (0, 0)))
    tgt_oh = jax.nn.one_hot(targets, _NA)

    def row(v):
        return v.reshape(1, -1)

    args = (
        xin, tgt_oh,
        preWin, row(pre_bin), pre_Wh[0], row(pre_bh[0]), pre_Wh[1],
        row(pre_bh[1]), pre_Wout, row(pre_bout),
        msg_Win[:2 * _H], row(msg_bin), msg_Wh[0], row(msg_bh[0]),
        msg_Wh[1], row(msg_bh[1]), msg_Wout, row(msg_bout),
        post_Win, row(post_bin), post_Wh[0], row(post_bh[0]), post_Wh[1],
        row(post_bh[1]), post_Wout, row(post_bout),
        out_Win, row(out_bin), out_Wh[0], row(out_bh[0]), out_Wh[1],
        row(out_bh[1]), out_Wout, row(out_bout),
        row(bn_gamma), row(bn_beta), lstm_W, row(lstm_b),
    )
    loss, idx = pl.pallas_call(
        _fwd_kernel,
        out_shape=[jax.ShapeDtypeStruct((_STEPS, 1), jnp.float32),
                   jax.ShapeDtypeStruct((_STEPS, _BS), jnp.int32)],
    )(*args)
    return loss[:, 0], idx


# speedup vs baseline: 20.2980x; 1.1334x over previous
import jax
import jax.numpy as jnp
import numpy as np
from jax.experimental import pallas as pl
from jax.experimental.pallas import tpu as pltpu

_BS = 512
_N = 8
_H = 128
_STEPS = 8
_NA = 16
_ROWS = _BS * _N
_DIN = 2 + 8 + 8 + _NA + _N


def _dot(a, b):
    return jax.lax.dot_general(a, b, (((1,), (0,)), ((), ())),
                               preferred_element_type=jnp.float32)


def _mlp_tail(x, W0, b0, W1, b1, Wout, bo):
    x = jnp.maximum(_dot(x, W0) + b0, 0.0)
    x = jnp.maximum(_dot(x, W1) + b1, 0.0)
    return _dot(x, Wout) + bo


def _mlp4(x, Win, bi, W0, b0, W1, b1, Wout, bo):
    x = jnp.maximum(_dot(x, Win) + bi, 0.0)
    return _mlp_tail(x, W0, b0, W1, b1, Wout, bo)


def _band(lane, off, vals):
    return jnp.where((lane - off) == vals, 1.0, 0.0)


def _fwd_kernel(px_ref, py_ref, col_ref, mar_ref, anc_ref, nj_ref, tgt_ref,
                preWin, prebin, preWh, prebh, preWout, prebout,
                msgWin, msgbin, msgWh, msgbh, msgWout, msgbout,
                postWin, postbin, postWh, postbh, postWout, postbout,
                outWin, outbin, outWh, outbh, outWout, outbout,
                gamma_ref, beta_ref, lstmW, lstmb,
                loss_ref, idx_ref, e_ref):
    lane = jax.lax.broadcasted_iota(jnp.int32, (_BS, _N, _H), 2)
    px = px_ref[...][:, :, None]
    py = py_ref[...][:, :, None]
    col = col_ref[...][:, :, None]
    mar = mar_ref[...][:, :, None]
    anc = anc_ref[...][:, :, None]
    nj = nj_ref[...][:, :, None]
    zero = jnp.zeros((), jnp.float32)
    xin = (jnp.where(lane == 0, px, zero) + jnp.where(lane == 1, py, zero)
           + _band(lane, 2, col) + _band(lane, 10, mar)
           + _band(lane, 18, anc) + _band(lane, 34, nj)).reshape(_ROWS, _H)
    preW = jnp.concatenate(
        [preWin[...], jnp.zeros((_H - _DIN, _H), jnp.float32)], axis=0)
    tgt = _band(jax.lax.broadcasted_iota(jnp.int32, (_BS, _NA), 1), 0,
                tgt_ref[...])

    x = _mlp4(xin, preW, prebin[...], preWh[0], prebh[0], preWh[1], prebh[1],
              preWout[...], prebout[...])
    x0 = x
    c = jnp.zeros((_ROWS, _H), jnp.float32)
    h = jnp.zeros((_ROWS, _H), jnp.float32)
    gamma = gamma_ref[...]
    beta = beta_ref[...]
    msgW = msgWin[0:2 * _H, :]
    losses = []
    idxs = []
    for _ in range(_STEPS):
        x3 = x.reshape(_BS, _N, _H)
        e_ref[:, _H:] = x
        xm = jnp.zeros((_ROWS, _H), jnp.float32)
        for i in range(_N):
            e_ref[:, :_H] = jnp.broadcast_to(
                x3[:, i, :][:, None, :], (_BS, _N, _H)).reshape(_ROWS, _H)
            m = _mlp4(e_ref[...], msgW, msgbin[...], msgWh[0], msgbh[0],
                      msgWh[1], msgbh[1], msgWout[...], msgbout[...])
            xm = xm + m
        t = jnp.concatenate([xm, x0], axis=1)
        x = _mlp4(t, postWin[...], postbin[...], postWh[0], postbh[0],
                  postWh[1], postbh[1], postWout[...], postbout[...])
        mean = jnp.mean(x, axis=0)
        var = jnp.var(x, axis=0)
        x = (x - mean) / jnp.sqrt(var + 1e-3) * gamma + beta
        gates = _dot(jnp.concatenate([x, h], axis=1), lstmW[...]) + lstmb[...]
        gi = gates[:, 0 * _H:1 * _H]
        gj = gates[:, 1 * _H:2 * _H]
        gf = gates[:, 2 * _H:3 * _H]
        go = gates[:, 3 * _H:4 * _H]
        c = jax.nn.sigmoid(gf + 1.0) * c + jax.nn.sigmoid(gi) * jnp.tanh(gj)
        h = jnp.tanh(c) * jax.nn.sigmoid(go)
        x = h
        xs = jnp.sum(x.reshape(_BS, _N, _H), axis=1)
        logits = _mlp4(xs, outWin[...], outbin[...], outWh[0], outbh[0],
                       outWh[1], outbh[1], outWout[...], outbout[...])
        mx = jnp.max(logits, axis=1, keepdims=True)
        ii = jax.lax.broadcasted_iota(jnp.int32, (_BS, _NA), 1)
        idxs.append(jnp.min(jnp.where(logits == mx, ii, _NA), axis=1))
        z = logits - mx
        lse = jnp.log(jnp.sum(jnp.exp(z), axis=1, keepdims=True))
        ce = -jnp.sum((z - lse) * tgt, axis=1)
        losses.append(jnp.mean(ce) / np.log(2.0))
    idx_ref[...] = jnp.stack(idxs, axis=0)
    loss_ref[...] = jnp.stack(losses).reshape(_STEPS, 1)


@jax.jit
def kernel(positions, colors, markers, anchors, n_jumps, targets,
           pre_Win, pre_bin, pre_Wh, pre_bh, pre_Wout, pre_bout,
           msg_Win, msg_bin, msg_Wh, msg_bh, msg_Wout, msg_bout,
           post_Win, post_bin, post_Wh, post_bh, post_Wout, post_bout,
           out_Win, out_bin, out_Wh, out_bh, out_Wout, out_bout,
           bn_gamma, bn_beta, lstm_W, lstm_b):
    args = (
        positions[:, :, 0], positions[:, :, 1], colors, markers,
        anchors.reshape(_BS, 1), n_jumps.reshape(_BS, 1),
        targets.reshape(_BS, 1),
        pre_Win, pre_bin, pre_Wh, pre_bh, pre_Wout, pre_bout,
        msg_Win, msg_bin, msg_Wh, msg_bh, msg_Wout, msg_bout,
        post_Win, post_bin, post_Wh, post_bh, post_Wout, post_bout,
        out_Win, out_bin, out_Wh, out_bh, out_Wout, out_bout,
        bn_gamma, bn_beta, lstm_W, lstm_b,
    )
    loss, idx = pl.pallas_call(
        _fwd_kernel,
        out_shape=[jax.ShapeDtypeStruct((_STEPS, 1), jnp.float32),
                   jax.ShapeDtypeStruct((_STEPS, _BS), jnp.int32)],
        scratch_shapes=[pltpu.VMEM((_ROWS, 2 * _H), jnp.float32)],
    )(*args)
    return loss[:, 0], idx


# speedup vs baseline: 20.3389x; 1.0020x over previous
import jax
import jax.numpy as jnp
import numpy as np
from jax.experimental import pallas as pl
from jax.experimental.pallas import tpu as pltpu

_BS = 512
_N = 8
_H = 128
_STEPS = 8
_NA = 16
_ROWS = _BS * _N
_DIN = 2 + 8 + 8 + _NA + _N


def _dot(a, b):
    return jax.lax.dot_general(a, b, (((1,), (0,)), ((), ())),
                               preferred_element_type=jnp.float32)


def _mlp_tail(x, W0, b0, W1, b1, Wout, bo):
    x = jnp.maximum(_dot(x, W0) + b0, 0.0)
    x = jnp.maximum(_dot(x, W1) + b1, 0.0)
    return _dot(x, Wout) + bo


def _mlp4(x, Win, bi, W0, b0, W1, b1, Wout, bo):
    x = jnp.maximum(_dot(x, Win) + bi, 0.0)
    return _mlp_tail(x, W0, b0, W1, b1, Wout, bo)


def _band(lane, off, vals):
    return jnp.where((lane - off) == vals, 1.0, 0.0)


def _fwd_kernel(px_ref, py_ref, col_ref, mar_ref, anc_ref, nj_ref, tgt_ref,
                preWin, prebin, preWh, prebh, preWout, prebout,
                msgWin, msgbin, msgWh, msgbh, msgWout, msgbout,
                postWin, postbin, postWh, postbh, postWout, postbout,
                outWin, outbin, outWh, outbh, outWout, outbout,
                gamma_ref, beta_ref, lstmW, lstmb,
                loss_ref, idx_ref, e_ref, e2_ref):
    lane = jax.lax.broadcasted_iota(jnp.int32, (_BS, _N, _H), 2)
    px = px_ref[...][:, :, None]
    py = py_ref[...][:, :, None]
    col = col_ref[...][:, :, None]
    mar = mar_ref[...][:, :, None]
    anc = anc_ref[...][:, :, None]
    nj = nj_ref[...][:, :, None]
    zero = jnp.zeros((), jnp.float32)
    xin = (jnp.where(lane == 0, px, zero) + jnp.where(lane == 1, py, zero)
           + _band(lane, 2, col) + _band(lane, 10, mar)
           + _band(lane, 18, anc) + _band(lane, 34, nj)).reshape(_ROWS, _H)
    preW = jnp.concatenate(
        [preWin[...], jnp.zeros((_H - _DIN, _H), jnp.float32)], axis=0)
    tgt = _band(jax.lax.broadcasted_iota(jnp.int32, (_BS, _NA), 1), 0,
                tgt_ref[...])

    x = _mlp4(xin, preW, prebin[...], preWh[0], prebh[0], preWh[1], prebh[1],
              preWout[...], prebout[...])
    x0 = x
    c = jnp.zeros((_ROWS, _H), jnp.float32)
    h = jnp.zeros((_ROWS, _H), jnp.float32)
    gamma = gamma_ref[...]
    beta = beta_ref[...]
    msgW = msgWin[0:2 * _H, :]
    losses = []
    idxs = []
    for _ in range(_STEPS):
        x3 = x.reshape(_BS, _N, _H)
        e_ref[:, _H:] = x
        e2_ref[:, _H:] = x
        xm = jnp.zeros((_ROWS, _H), jnp.float32)
        for i in range(_N):
            buf = e_ref if i % 2 == 0 else e2_ref
            buf[:, :_H] = jnp.broadcast_to(
                x3[:, i, :][:, None, :], (_BS, _N, _H)).reshape(_ROWS, _H)
            m = _mlp4(buf[...], msgW, msgbin[...], msgWh[0], msgbh[0],
                      msgWh[1], msgbh[1], msgWout[...], msgbout[...])
            xm = xm + m
        t = jnp.concatenate([xm, x0], axis=1)
        x = _mlp4(t, postWin[...], postbin[...], postWh[0], postbh[0],
                  postWh[1], postbh[1], postWout[...], postbout[...])
        mean = jnp.mean(x, axis=0)
        var = jnp.var(x, axis=0)
        x = (x - mean) / jnp.sqrt(var + 1e-3) * gamma + beta
        gates = _dot(jnp.concatenate([x, h], axis=1), lstmW[...]) + lstmb[...]
        gi = gates[:, 0 * _H:1 * _H]
        gj = gates[:, 1 * _H:2 * _H]
        gf = gates[:, 2 * _H:3 * _H]
        go = gates[:, 3 * _H:4 * _H]
        c = jax.nn.sigmoid(gf + 1.0) * c + jax.nn.sigmoid(gi) * jnp.tanh(gj)
        h = jnp.tanh(c) * jax.nn.sigmoid(go)
        x = h
        xs = jnp.sum(x.reshape(_BS, _N, _H), axis=1)
        logits = _mlp4(xs, outWin[...], outbin[...], outWh[0], outbh[0],
                       outWh[1], outbh[1], outWout[...], outbout[...])
        mx = jnp.max(logits, axis=1, keepdims=True)
        ii = jax.lax.broadcasted_iota(jnp.int32, (_BS, _NA), 1)
        idxs.append(jnp.min(jnp.where(logits == mx, ii, _NA), axis=1))
        z = logits - mx
        lse = jnp.log(jnp.sum(jnp.exp(z), axis=1, keepdims=True))
        ce = -jnp.sum((z - lse) * tgt, axis=1)
        losses.append(jnp.mean(ce) / np.log(2.0))
    idx_ref[...] = jnp.stack(idxs, axis=0)
    loss_ref[...] = jnp.stack(losses).reshape(_STEPS, 1)


@jax.jit
def kernel(positions, colors, markers, anchors, n_jumps, targets,
           pre_Win, pre_bin, pre_Wh, pre_bh, pre_Wout, pre_bout,
           msg_Win, msg_bin, msg_Wh, msg_bh, msg_Wout, msg_bout,
           post_Win, post_bin, post_Wh, post_bh, post_Wout, post_bout,
           out_Win, out_bin, out_Wh, out_bh, out_Wout, out_bout,
           bn_gamma, bn_beta, lstm_W, lstm_b):
    args = (
        positions[:, :, 0], positions[:, :, 1], colors, markers,
        anchors.reshape(_BS, 1), n_jumps.reshape(_BS, 1),
        targets.reshape(_BS, 1),
        pre_Win, pre_bin, pre_Wh, pre_bh, pre_Wout, pre_bout,
        msg_Win, msg_bin, msg_Wh, msg_bh, msg_Wout, msg_bout,
        post_Win, post_bin, post_Wh, post_bh, post_Wout, post_bout,
        out_Win, out_bin, out_Wh, out_bh, out_Wout, out_bout,
        bn_gamma, bn_beta, lstm_W, lstm_b,
    )
    loss, idx = pl.pallas_call(
        _fwd_kernel,
        out_shape=[jax.ShapeDtypeStruct((_STEPS, 1), jnp.float32),
                   jax.ShapeDtypeStruct((_STEPS, _BS), jnp.int32)],
        scratch_shapes=[pltpu.VMEM((_ROWS, 2 * _H), jnp.float32),
                        pltpu.VMEM((_ROWS, 2 * _H), jnp.float32)],
    )(*args)
    return loss[:, 0], idx
